# Initial kernel scaffold; baseline (speedup 1.0000x reference)
import jax
import jax.numpy as jnp
from jax.experimental import pallas as pl


def kernel(x, batch, W_ih, W_hh, b_ih, b_hh):
    raise NotImplementedError("write your pallas kernel here")


# speedup vs baseline: 17.6173x; 17.6173x over previous
import functools

import jax
import jax.numpy as jnp
from jax.experimental import pallas as pl
from jax.experimental.pallas import tpu as pltpu

N = 100000
D = 128
B = 256
STEPS = 3
R = 2000
NB = N // R

_NEG = -1e30


def _set2set_kernel(x_ref, b_ref, wih_ref, whh_ref, bih_ref, bhh_ref,
                    out_ref, e_scr, h_scr, c_scr, qs_scr, m_scr, d_scr, r_scr):
    f32 = jnp.float32

    h_scr[...] = jnp.zeros((B, D), f32)
    c_scr[...] = jnp.zeros((B, D), f32)
    qs_scr[...] = jnp.zeros((B, 2 * D), f32)

    seg_ids = jax.lax.broadcasted_iota(jnp.int32, (B, 1), 0)

    for _ in range(STEPS):
        qs = qs_scr[...]
        h = h_scr[...]
        c = c_scr[...]
        gates = (jax.lax.dot_general(qs, wih_ref[...],
                                     (((1,), (1,)), ((), ())),
                                     preferred_element_type=f32)
                 + jax.lax.dot_general(h, whh_ref[...],
                                       (((1,), (1,)), ((), ())),
                                       preferred_element_type=f32)
                 + bih_ref[...] + bhh_ref[...])
        ig = jax.nn.sigmoid(gates[:, 0 * D:1 * D])
        fg = jax.nn.sigmoid(gates[:, 1 * D:2 * D])
        gg = jnp.tanh(gates[:, 2 * D:3 * D])
        og = jax.nn.sigmoid(gates[:, 3 * D:4 * D])
        c = fg * c + ig * gg
        h = og * jnp.tanh(c)
        h_scr[...] = h
        c_scr[...] = c

        m_scr[...] = jnp.full((B, 128), _NEG, f32)
        d_scr[...] = jnp.zeros((B, 128), f32)

        def pass1(blk, _):
            xb = x_ref[pl.ds(blk * R, R), :]
            bb = b_ref[pl.ds(blk, 1), :]
            oh = bb == seg_ids
            E = jax.lax.dot_general(h, xb, (((1,), (1,)), ((), ())),
                                    preferred_element_type=f32)
            Em = jnp.where(oh, E, _NEG)
            bm = jnp.max(Em, axis=1, keepdims=True)
            m_old = m_scr[:, 0:1]
            m_new = jnp.maximum(m_old, bm)
            scale = jnp.exp(m_old - m_new)
            dd = jnp.sum(jnp.exp(Em - m_new), axis=1, keepdims=True)
            d_scr[:, 0:1] = d_scr[:, 0:1] * scale + dd
            m_scr[:, 0:1] = m_new
            e_scr[pl.ds(blk, 1), :] = jnp.sum(
                jnp.where(oh, E, 0.0), axis=0, keepdims=True)
            return 0

        jax.lax.fori_loop(0, NB, pass1, 0)

        r_scr[...] = jnp.zeros((B, D), f32)
        m_col = m_scr[:, 0:1]
        d_col = d_scr[:, 0:1]

        def pass2(blk, _):
            xb = x_ref[pl.ds(blk * R, R), :]
            bb = b_ref[pl.ds(blk, 1), :]
            oh = bb == seg_ids
            ohf = oh.astype(f32)
            e = e_scr[pl.ds(blk, 1), :]
            mg = jnp.sum(ohf * m_col, axis=0, keepdims=True)
            dg = jnp.sum(ohf * d_col, axis=0, keepdims=True)
            a = jnp.exp(e - mg) / (dg + 1e-16)
            W = ohf * a
            r_scr[...] += jax.lax.dot_general(
                W, xb, (((1,), (0,)), ((), ())),
                preferred_element_type=f32)
            return 0

        jax.lax.fori_loop(0, NB, pass2, 0)

        qs_scr[:, 0:D] = h
        qs_scr[:, D:2 * D] = r_scr[...]

    out_ref[...] = qs_scr[...]


@jax.jit
def kernel(x, batch, W_ih, W_hh, b_ih, b_hh):
    batch2d = batch.astype(jnp.int32).reshape(NB, R)
    bih2d = b_ih.reshape(1, 4 * D)
    bhh2d = b_hh.reshape(1, 4 * D)
    out = pl.pallas_call(
        _set2set_kernel,
        in_specs=[pl.BlockSpec(memory_space=pltpu.VMEM)] * 6,
        out_specs=pl.BlockSpec(memory_space=pltpu.VMEM),
        out_shape=jax.ShapeDtypeStruct((B, 2 * D), jnp.float32),
        scratch_shapes=[
            pltpu.VMEM((NB, R), jnp.float32),
            pltpu.VMEM((B, D), jnp.float32),
            pltpu.VMEM((B, D), jnp.float32),
            pltpu.VMEM((B, 2 * D), jnp.float32),
            pltpu.VMEM((B, 128), jnp.float32),
            pltpu.VMEM((B, 128), jnp.float32),
            pltpu.VMEM((B, D), jnp.float32),
        ],
        compiler_params=pltpu.CompilerParams(
            vmem_limit_bytes=100 * 1024 * 1024,
        ),
    )(x, batch2d, W_ih, W_hh, bih2d, bhh2d)
    return out


# speedup vs baseline: 34.8418x; 1.9777x over previous
import jax
import jax.numpy as jnp
from jax.experimental import pallas as pl
from jax.experimental.pallas import tpu as pltpu

N = 100000
D = 128
B = 256
STEPS = 3
R = 2000
NB = N // R
W = 32

_NEG = -1e30


def _set2set_kernel(x_ref, b_ref, lo_ref, nw_ref,
                    wih_ref, whh_ref, bih_ref, bhh_ref,
                    out_ref, e_scr, h_scr, c_scr, qs_scr, m_scr, d_scr, u_scr):
    f32 = jnp.float32

    h_scr[...] = jnp.zeros((B, D), f32)
    c_scr[...] = jnp.zeros((B, D), f32)
    qs_scr[...] = jnp.zeros((B, 2 * D), f32)

    for _ in range(STEPS):
        qs = qs_scr[...]
        h = h_scr[...]
        c = c_scr[...]
        gates = (jax.lax.dot_general(qs, wih_ref[...],
                                     (((1,), (1,)), ((), ())),
                                     preferred_element_type=f32)
                 + jax.lax.dot_general(h, whh_ref[...],
                                       (((1,), (1,)), ((), ())),
                                       preferred_element_type=f32)
                 + bih_ref[...] + bhh_ref[...])
        ig = jax.nn.sigmoid(gates[:, 0 * D:1 * D])
        fg = jax.nn.sigmoid(gates[:, 1 * D:2 * D])
        gg = jnp.tanh(gates[:, 2 * D:3 * D])
        og = jax.nn.sigmoid(gates[:, 3 * D:4 * D])
        c = fg * c + ig * gg
        h = og * jnp.tanh(c)
        h_scr[...] = h
        c_scr[...] = c

        m_scr[...] = jnp.full((B, 128), _NEG, f32)
        d_scr[...] = jnp.zeros((B, 128), f32)

        def p1_body(blk, lo, w):
            xb = x_ref[pl.ds(blk * R, R), :]
            bb = b_ref[pl.ds(blk, 1), :]
            oh = (bb - lo) == jax.lax.broadcasted_iota(
                jnp.int32, (w, 1), 0)
            qw = h_scr[pl.ds(lo, w), :]
            E = jax.lax.dot_general(qw, xb, (((1,), (1,)), ((), ())),
                                    preferred_element_type=f32)
            Em = jnp.where(oh, E, _NEG)
            bm = jnp.max(Em, axis=1, keepdims=True)
            m_old = m_scr[pl.ds(lo, w), 0:1]
            m_new = jnp.maximum(m_old, bm)
            scale = jnp.exp(m_old - m_new)
            dd = jnp.sum(jnp.exp(Em - m_new), axis=1, keepdims=True)
            d_scr[pl.ds(lo, w), 0:1] = d_scr[pl.ds(lo, w), 0:1] * scale + dd
            m_scr[pl.ds(lo, w), 0:1] = m_new
            e_scr[pl.ds(blk, 1), :] = jnp.sum(
                jnp.where(oh, E, 0.0), axis=0, keepdims=True)

        def pass1(blk, _):
            @pl.when(nw_ref[blk] == 1)
            def _narrow():
                p1_body(blk, lo_ref[blk], W)

            @pl.when(nw_ref[blk] == 0)
            def _full():
                p1_body(blk, 0, B)
            return 0

        jax.lax.fori_loop(0, NB, pass1, 0)

        u_scr[...] = jnp.zeros((B, D), f32)

        def p2_body(blk, lo, w):
            xb = x_ref[pl.ds(blk * R, R), :]
            bb = b_ref[pl.ds(blk, 1), :]
            ohf = ((bb - lo) == jax.lax.broadcasted_iota(
                jnp.int32, (w, 1), 0)).astype(f32)
            m_w = m_scr[pl.ds(lo, w), 0:1]
            mg = jnp.sum(ohf * m_w, axis=0, keepdims=True)
            ex = jnp.exp(e_scr[pl.ds(blk, 1), :] - mg)
            Wm = ohf * ex
            u_scr[pl.ds(lo, w), :] += jax.lax.dot_general(
                Wm, xb, (((1,), (0,)), ((), ())),
                preferred_element_type=f32)

        def pass2(blk, _):
            @pl.when(nw_ref[blk] == 1)
            def _narrow():
                p2_body(blk, lo_ref[blk], W)

            @pl.when(nw_ref[blk] == 0)
            def _full():
                p2_body(blk, 0, B)
            return 0

        jax.lax.fori_loop(0, NB, pass2, 0)

        qs_scr[:, 0:D] = h
        qs_scr[:, D:2 * D] = u_scr[...] / (d_scr[:, 0:1] + 1e-16)

    out_ref[...] = qs_scr[...]


@jax.jit
def kernel(x, batch, W_ih, W_hh, b_ih, b_hh):
    batch2d = batch.astype(jnp.int32).reshape(NB, R)
    first = batch2d[:, 0]
    last = batch2d[:, -1]
    lo = jnp.minimum(jnp.bitwise_and(first, -8), B - W)
    narrow = (last - lo < W).astype(jnp.int32)
    bih2d = b_ih.reshape(1, 4 * D)
    bhh2d = b_hh.reshape(1, 4 * D)
    out = pl.pallas_call(
        _set2set_kernel,
        in_specs=[
            pl.BlockSpec(memory_space=pltpu.VMEM),
            pl.BlockSpec(memory_space=pltpu.VMEM),
            pl.BlockSpec(memory_space=pltpu.SMEM),
            pl.BlockSpec(memory_space=pltpu.SMEM),
            pl.BlockSpec(memory_space=pltpu.VMEM),
            pl.BlockSpec(memory_space=pltpu.VMEM),
            pl.BlockSpec(memory_space=pltpu.VMEM),
            pl.BlockSpec(memory_space=pltpu.VMEM),
        ],
        out_specs=pl.BlockSpec(memory_space=pltpu.VMEM),
        out_shape=jax.ShapeDtypeStruct((B, 2 * D), jnp.float32),
        scratch_shapes=[
            pltpu.VMEM((NB, R), jnp.float32),
            pltpu.VMEM((B, D), jnp.float32),
            pltpu.VMEM((B, D), jnp.float32),
            pltpu.VMEM((B, 2 * D), jnp.float32),
            pltpu.VMEM((B, 128), jnp.float32),
            pltpu.VMEM((B, 128), jnp.float32),
            pltpu.VMEM((B, D), jnp.float32),
        ],
        compiler_params=pltpu.CompilerParams(
            vmem_limit_bytes=100 * 1024 * 1024,
        ),
    )(x, batch2d, lo, narrow, W_ih, W_hh, bih2d, bhh2d)
    return out


# speedup vs baseline: 48.5242x; 1.3927x over previous
import jax
import jax.numpy as jnp
from jax.experimental import pallas as pl
from jax.experimental.pallas import tpu as pltpu

N = 100000
D = 128
B = 256
STEPS = 3
R = 4000
NB = N // R
W = 32

_NEG = -1e30


def _set2set_kernel(x_ref, b_ref, lo_ref, nw_ref,
                    wih_ref, whh_ref, bih_ref, bhh_ref,
                    out_ref, e_scr, h_scr, c_scr, qs_scr, m_scr, d_scr, u_scr):
    f32 = jnp.float32

    h_scr[...] = jnp.zeros((B, D), f32)
    c_scr[...] = jnp.zeros((B, D), f32)
    qs_scr[...] = jnp.zeros((B, 2 * D), f32)

    for _ in range(STEPS):
        qs = qs_scr[...]
        h = h_scr[...]
        c = c_scr[...]
        gates = (jax.lax.dot_general(qs, wih_ref[...],
                                     (((1,), (1,)), ((), ())),
                                     preferred_element_type=f32)
                 + jax.lax.dot_general(h, whh_ref[...],
                                       (((1,), (1,)), ((), ())),
                                       preferred_element_type=f32)
                 + bih_ref[...] + bhh_ref[...])
        ig = jax.nn.sigmoid(gates[:, 0 * D:1 * D])
        fg = jax.nn.sigmoid(gates[:, 1 * D:2 * D])
        gg = jnp.tanh(gates[:, 2 * D:3 * D])
        og = jax.nn.sigmoid(gates[:, 3 * D:4 * D])
        c = fg * c + ig * gg
        h = og * jnp.tanh(c)
        h_scr[...] = h
        c_scr[...] = c

        m_scr[...] = jnp.full((B, 128), _NEG, f32)
        d_scr[...] = jnp.zeros((B, 128), f32)

        def p1_body(blk, lo, w):
            xb = x_ref[pl.ds(blk * R, R), :]
            bb = b_ref[pl.ds(blk, 1), :]
            oh = (bb - lo) == jax.lax.broadcasted_iota(
                jnp.int32, (w, 1), 0)
            qw = h_scr[pl.ds(lo, w), :]
            E = jax.lax.dot_general(qw, xb, (((1,), (1,)), ((), ())),
                                    preferred_element_type=f32)
            Em = jnp.where(oh, E, _NEG)
            bm = jnp.max(Em, axis=1, keepdims=True)
            m_old = m_scr[pl.ds(lo, w), 0:1]
            m_new = jnp.maximum(m_old, bm)
            scale = jnp.exp(m_old - m_new)
            dd = jnp.sum(jnp.exp(Em - m_new), axis=1, keepdims=True)
            d_scr[pl.ds(lo, w), 0:1] = d_scr[pl.ds(lo, w), 0:1] * scale + dd
            m_scr[pl.ds(lo, w), 0:1] = m_new
            e_scr[pl.ds(blk, 1), :] = jnp.sum(
                jnp.where(oh, E, 0.0), axis=0, keepdims=True)

        def pass1(blk, _):
            @pl.when(nw_ref[blk] == 1)
            def _narrow():
                p1_body(blk, lo_ref[blk], W)

            @pl.when(nw_ref[blk] == 0)
            def _full():
                p1_body(blk, 0, B)
            return 0

        jax.lax.fori_loop(0, NB, pass1, 0)

        u_scr[...] = jnp.zeros((B, D), f32)

        def p2_body(blk, lo, w):
            xb = x_ref[pl.ds(blk * R, R), :]
            bb = b_ref[pl.ds(blk, 1), :]
            ohf = ((bb - lo) == jax.lax.broadcasted_iota(
                jnp.int32, (w, 1), 0)).astype(f32)
            m_w = m_scr[pl.ds(lo, w), 0:1]
            mg = jnp.sum(ohf * m_w, axis=0, keepdims=True)
            ex = jnp.exp(e_scr[pl.ds(blk, 1), :] - mg)
            Wm = ohf * ex
            u_scr[pl.ds(lo, w), :] += jax.lax.dot_general(
                Wm, xb, (((1,), (0,)), ((), ())),
                preferred_element_type=f32)

        def pass2(blk, _):
            @pl.when(nw_ref[blk] == 1)
            def _narrow():
                p2_body(blk, lo_ref[blk], W)

            @pl.when(nw_ref[blk] == 0)
            def _full():
                p2_body(blk, 0, B)
            return 0

        jax.lax.fori_loop(0, NB, pass2, 0)

        qs_scr[:, 0:D] = h
        qs_scr[:, D:2 * D] = u_scr[...] / (d_scr[:, 0:1] + 1e-16)

    out_ref[...] = qs_scr[...]


@jax.jit
def kernel(x, batch, W_ih, W_hh, b_ih, b_hh):
    batch2d = batch.astype(jnp.int32).reshape(NB, R)
    first = batch2d[:, 0]
    last = batch2d[:, -1]
    lo = jnp.minimum(jnp.bitwise_and(first, -8), B - W)
    narrow = (last - lo < W).astype(jnp.int32)
    bih2d = b_ih.reshape(1, 4 * D)
    bhh2d = b_hh.reshape(1, 4 * D)
    out = pl.pallas_call(
        _set2set_kernel,
        in_specs=[
            pl.BlockSpec(memory_space=pltpu.VMEM),
            pl.BlockSpec(memory_space=pltpu.VMEM),
            pl.BlockSpec(memory_space=pltpu.SMEM),
            pl.BlockSpec(memory_space=pltpu.SMEM),
            pl.BlockSpec(memory_space=pltpu.VMEM),
            pl.BlockSpec(memory_space=pltpu.VMEM),
            pl.BlockSpec(memory_space=pltpu.VMEM),
            pl.BlockSpec(memory_space=pltpu.VMEM),
        ],
        out_specs=pl.BlockSpec(memory_space=pltpu.VMEM),
        out_shape=jax.ShapeDtypeStruct((B, 2 * D), jnp.float32),
        scratch_shapes=[
            pltpu.VMEM((NB, R), jnp.float32),
            pltpu.VMEM((B, D), jnp.float32),
            pltpu.VMEM((B, D), jnp.float32),
            pltpu.VMEM((B, 2 * D), jnp.float32),
            pltpu.VMEM((B, 128), jnp.float32),
            pltpu.VMEM((B, 128), jnp.float32),
            pltpu.VMEM((B, D), jnp.float32),
        ],
        compiler_params=pltpu.CompilerParams(
            vmem_limit_bytes=100 * 1024 * 1024,
        ),
    )(x, batch2d, lo, narrow, W_ih, W_hh, bih2d, bhh2d)
    return out
